# Initial kernel scaffold; baseline (speedup 1.0000x reference)
import jax
import jax.numpy as jnp
from jax.experimental import pallas as pl


def kernel(x, enc_W, enc_b, W_dec, b_dec):
    raise NotImplementedError("write your pallas kernel here")


# speedup vs baseline: 2.4211x; 2.4211x over previous
import functools

import jax
import jax.numpy as jnp
from jax import lax
from jax.experimental import pallas as pl
from jax.experimental.pallas import tpu as pltpu
from jax.experimental.pallas import tpu_sc as plsc

_S, _D, _L, _K = 2048, 1024, 8192, 32
_BS = 256
_BL = 512

_NC, _NS = 2, 16
_NW = _NC * _NS
_TPW = _S // _NW


def _encode_topk_body(x_ref, bdec_ref, w_ref, b_ref, vals_ref, idx_ref, pre_ref):
    j = pl.program_id(1)
    sae = x_ref[...] - bdec_ref[...]
    z = lax.dot_general(sae, w_ref[...], (((1,), (1,)), ((), ())),
                        preferred_element_type=jnp.float32)
    z = jnp.maximum(z + b_ref[...], 0.0)
    pre_ref[:, pl.ds(j * _BL, _BL)] = z

    @pl.when(j == pl.num_programs(1) - 1)
    def _():
        iota = lax.broadcasted_iota(jnp.int32, (_BS, _L), 1)
        neg = jnp.float32(-jnp.inf)
        for k in range(_K):
            buf = pre_ref[...]
            m = jnp.max(buf, axis=1)
            eq = buf == m[:, None]
            idx = jnp.min(jnp.where(eq, iota, _L), axis=1)
            vals_ref[:, k:k + 1] = m[:, None]
            idx_ref[:, k:k + 1] = idx[:, None]
            pre_ref[...] = jnp.where(iota == idx[:, None], neg, buf)


_encode_topk = pl.pallas_call(
    _encode_topk_body,
    grid=(_S // _BS, _L // _BL),
    in_specs=[
        pl.BlockSpec((_BS, _D), lambda i, j: (i, 0)),
        pl.BlockSpec((1, _D), lambda i, j: (0, 0)),
        pl.BlockSpec((_BL, _D), lambda i, j: (j, 0)),
        pl.BlockSpec((1, _BL), lambda i, j: (0, j)),
    ],
    out_specs=[
        pl.BlockSpec((_BS, _K), lambda i, j: (i, 0)),
        pl.BlockSpec((_BS, _K), lambda i, j: (i, 0)),
    ],
    out_shape=[
        jax.ShapeDtypeStruct((_S, _K), jnp.float32),
        jax.ShapeDtypeStruct((_S, _K), jnp.int32),
    ],
    scratch_shapes=[pltpu.VMEM((_BS, _L), jnp.float32)],
)


def _sc_decode_body(idx_hbm, vals_hbm, wdec_hbm, bdec_hbm, out_hbm,
                    idx_v, vals_v, rows_v, bdec_v, acc_v, sem):
    wid = lax.axis_index("s") * _NC + lax.axis_index("c")
    base = wid * _TPW
    pltpu.sync_copy(idx_hbm.at[pl.ds(base, _TPW)], idx_v)
    pltpu.sync_copy(vals_hbm.at[pl.ds(base, _TPW)], vals_v)
    pltpu.sync_copy(bdec_hbm, bdec_v)

    def token_body(t, carry):
        cp = pltpu.async_copy(wdec_hbm.at[idx_v.at[t]], rows_v, sem)
        for c in range(_D // 16):
            acc_v[pl.ds(c * 16, 16)] = bdec_v[pl.ds(c * 16, 16)]
        cp.wait()

        def k_body(k, carry2):
            kc = (k // 16) * 16
            vch = vals_v[t, pl.ds(kc, 16)]
            lane = jnp.broadcast_to(k % 16, (16,)).astype(jnp.int32)
            vb = vch[lane]
            for c in range(_D // 16):
                plsc.addupdate(acc_v.at[pl.ds(c * 16, 16)],
                               vb * rows_v[k, pl.ds(c * 16, 16)])
            return carry2

        lax.fori_loop(0, _K, k_body, 0)
        pltpu.sync_copy(acc_v, out_hbm.at[base + t])
        return carry

    lax.fori_loop(0, _TPW, token_body, 0)


@functools.cache
def _make_sc_decode():
    return pl.kernel(
        _sc_decode_body,
        mesh=plsc.VectorSubcoreMesh(core_axis_name="c", subcore_axis_name="s"),
        out_type=jax.ShapeDtypeStruct((_S, _D), jnp.float32),
        scratch_types=[
            pltpu.VMEM((_TPW, _K), jnp.int32),
            pltpu.VMEM((_TPW, _K), jnp.float32),
            pltpu.VMEM((_K, _D), jnp.float32),
            pltpu.VMEM((_D,), jnp.float32),
            pltpu.VMEM((_D,), jnp.float32),
            pltpu.SemaphoreType.DMA,
        ],
    )


def kernel(x, enc_W, enc_b, W_dec, b_dec):
    xs = x.reshape(_S, _D)
    vals, idxs = _encode_topk(xs, b_dec.reshape(1, _D), enc_W,
                              enc_b.reshape(1, _L))
    out = _make_sc_decode()(idxs, vals, W_dec, b_dec)
    return out.reshape(x.shape)


# speedup vs baseline: 3.8363x; 1.5845x over previous
import functools

import jax
import jax.numpy as jnp
from jax import lax
from jax.experimental import pallas as pl
from jax.experimental.pallas import tpu as pltpu
from jax.experimental.pallas import tpu_sc as plsc

_S, _D, _L, _K = 2048, 1024, 8192, 32
_BS = 256
_BL = 512

_NC, _NS = 2, 16
_NW = _NC * _NS
_TPW = _S // _NW


def _encode_topk_body(x_ref, bdec_ref, w_ref, b_ref, vals_ref, idx_ref, pre_ref):
    j = pl.program_id(1)
    sae = x_ref[...] - bdec_ref[...]
    z = lax.dot_general(sae, w_ref[...], (((1,), (1,)), ((), ())),
                        preferred_element_type=jnp.float32)
    z = jnp.maximum(z + b_ref[...], 0.0)
    pre_ref[:, pl.ds(j * _BL, _BL)] = z

    @pl.when(j == pl.num_programs(1) - 1)
    def _():
        iota = lax.broadcasted_iota(jnp.int32, (_BS, _L), 1)
        neg = jnp.float32(-jnp.inf)
        for k in range(_K):
            buf = pre_ref[...]
            m = jnp.max(buf, axis=1)
            eq = buf == m[:, None]
            idx = jnp.min(jnp.where(eq, iota, _L), axis=1)
            vals_ref[:, k:k + 1] = m[:, None]
            idx_ref[:, k:k + 1] = idx[:, None]
            pre_ref[...] = jnp.where(iota == idx[:, None], neg, buf)


_encode_topk = pl.pallas_call(
    _encode_topk_body,
    grid=(_S // _BS, _L // _BL),
    in_specs=[
        pl.BlockSpec((_BS, _D), lambda i, j: (i, 0)),
        pl.BlockSpec((1, _D), lambda i, j: (0, 0)),
        pl.BlockSpec((_BL, _D), lambda i, j: (j, 0)),
        pl.BlockSpec((1, _BL), lambda i, j: (0, j)),
    ],
    out_specs=[
        pl.BlockSpec((_BS, _K), lambda i, j: (i, 0)),
        pl.BlockSpec((_BS, _K), lambda i, j: (i, 0)),
    ],
    out_shape=[
        jax.ShapeDtypeStruct((_S, _K), jnp.float32),
        jax.ShapeDtypeStruct((_S, _K), jnp.int32),
    ],
    scratch_shapes=[pltpu.VMEM((_BS, _L), jnp.float32)],
)


def _sc_decode_body(idx_hbm, vals_hbm, wdec_hbm, bdec_hbm, out_hbm,
                    idx_v, vals_v, rows_v, bdec_v, acc_v, sem):
    wid = lax.axis_index("s") * _NC + lax.axis_index("c")
    base = wid * _TPW
    pltpu.sync_copy(idx_hbm.at[pl.ds(base, _TPW)], idx_v)
    pltpu.sync_copy(vals_hbm.at[pl.ds(base, _TPW)], vals_v)
    pltpu.sync_copy(bdec_hbm, bdec_v)

    pltpu.async_copy(wdec_hbm.at[idx_v.at[0]], rows_v.at[0], sem.at[0])

    def token_body(t, carry):
        b = lax.rem(t, 2)

        @pl.when(t + 1 < _TPW)
        def _():
            pltpu.async_copy(wdec_hbm.at[idx_v.at[t + 1]], rows_v.at[1 - b],
                             sem.at[1 - b])

        pltpu.make_async_copy(wdec_hbm.at[pl.ds(0, _K)], rows_v.at[b],
                              sem.at[b]).wait()

        vch0 = vals_v[t, pl.ds(0, 16)]
        vch1 = vals_v[t, pl.ds(16, 16)]
        vbs = ([vch0[jnp.full((16,), i, jnp.int32)] for i in range(16)] +
               [vch1[jnp.full((16,), i, jnp.int32)] for i in range(16)])

        def c_body(c, carry2):
            off = c * 32
            a0 = bdec_v[pl.ds(off, 16)]
            b0 = bdec_v[pl.ds(off + 16, 16)]
            a1 = jnp.zeros((16,), jnp.float32)
            b1 = jnp.zeros((16,), jnp.float32)
            for k in range(_K):
                wa = rows_v[b, k, pl.ds(off, 16)]
                wb = rows_v[b, k, pl.ds(off + 16, 16)]
                if k % 2 == 0:
                    a0 = a0 + vbs[k] * wa
                    b0 = b0 + vbs[k] * wb
                else:
                    a1 = a1 + vbs[k] * wa
                    b1 = b1 + vbs[k] * wb
            acc_v[pl.ds(off, 16)] = a0 + a1
            acc_v[pl.ds(off + 16, 16)] = b0 + b1
            return carry2

        lax.fori_loop(0, _D // 32, c_body, 0)
        pltpu.sync_copy(acc_v, out_hbm.at[base + t])
        return carry

    lax.fori_loop(0, _TPW, token_body, 0)


@functools.cache
def _make_sc_decode():
    return pl.kernel(
        _sc_decode_body,
        mesh=plsc.VectorSubcoreMesh(core_axis_name="c", subcore_axis_name="s"),
        out_type=jax.ShapeDtypeStruct((_S, _D), jnp.float32),
        scratch_types=[
            pltpu.VMEM((_TPW, _K), jnp.int32),
            pltpu.VMEM((_TPW, _K), jnp.float32),
            pltpu.VMEM((2, _K, _D), jnp.float32),
            pltpu.VMEM((_D,), jnp.float32),
            pltpu.VMEM((_D,), jnp.float32),
            pltpu.SemaphoreType.DMA((2,)),
        ],
    )


import numpy as _np
_PERM = _np.arange(_D).reshape(-1, 2, 16).transpose(0, 2, 1).reshape(-1)


def kernel(x, enc_W, enc_b, W_dec, b_dec):
    xs = x.reshape(_S, _D)
    vals, idxs = _encode_topk(xs, b_dec.reshape(1, _D), enc_W,
                              enc_b.reshape(1, _L))
    out = _make_sc_decode()(idxs, vals, W_dec, b_dec)
    return out.reshape(x.shape)


# speedup vs baseline: 5.5948x; 1.4584x over previous
import functools

import jax
import jax.numpy as jnp
from jax import lax
from jax.experimental import pallas as pl
from jax.experimental.pallas import tpu as pltpu
from jax.experimental.pallas import tpu_sc as plsc

_S, _D, _L, _K = 2048, 1024, 8192, 32
_BS = 256
_BL = 2048
_NCH = _L // 16
_NG = _L // 128
_NCAND = _K * 16

_NC, _NS = 2, 16
_NW = _NC * _NS
_TPW = _S // _NW


def _encode_body(x_ref, bdec_ref, w_ref, b_ref, pre_ref, cidx_ref, cm_ref):
    j = pl.program_id(1)
    sae = x_ref[...] - bdec_ref[...]
    z = lax.dot_general(sae, w_ref[...], (((1,), (1,)), ((), ())),
                        preferred_element_type=jnp.float32)
    z = jnp.maximum(z + b_ref[...], 0.0)
    pre_ref[...] = z.reshape(_BS, _BL // 128, 128)
    cmj = jnp.max(z.reshape(_BS, _BL // 16, 16), axis=2)
    cm_ref[:, pl.ds(j * (_BL // 16), _BL // 16)] = cmj

    @pl.when(j == pl.num_programs(1) - 1)
    def _():
        iota = lax.broadcasted_iota(jnp.int32, (_BS, _NCH), 1)
        neg = jnp.float32(-jnp.inf)
        for k in range(_K):
            buf = cm_ref[...]
            m = jnp.max(buf, axis=1)
            eq = buf == m[:, None]
            idx = jnp.min(jnp.where(eq, iota, _NCH), axis=1)
            cidx_ref[:, k:k + 1] = idx[:, None]
            cm_ref[...] = jnp.where(iota == idx[:, None], neg, buf)


_encode = pl.pallas_call(
    _encode_body,
    grid=(_S // _BS, _L // _BL),
    in_specs=[
        pl.BlockSpec((_BS, _D), lambda i, j: (i, 0)),
        pl.BlockSpec((1, _D), lambda i, j: (0, 0)),
        pl.BlockSpec((_BL, _D), lambda i, j: (j, 0)),
        pl.BlockSpec((1, _BL), lambda i, j: (0, j)),
    ],
    out_specs=[
        pl.BlockSpec((_BS, _BL // 128, 128), lambda i, j: (i, j, 0)),
        pl.BlockSpec((_BS, _K), lambda i, j: (i, 0)),
    ],
    out_shape=[
        jax.ShapeDtypeStruct((_S, _NG, 128), jnp.float32),
        jax.ShapeDtypeStruct((_S, _K), jnp.int32),
    ],
    scratch_shapes=[pltpu.VMEM((_BS, _NCH), jnp.float32)],
)


def _gather_body(pre_hbm, cidx_hbm, eye_hbm, cand_hbm, gi_hbm,
                 cidx_v, ggidx_v, grp_v, cand_v, gi_v, eye_v, csem):
    wid = lax.axis_index("s") * _NC + lax.axis_index("c")
    base = wid * _TPW
    pltpu.sync_copy(cidx_hbm.at[pl.ds(base, _TPW)], cidx_v)
    pltpu.sync_copy(eye_hbm, eye_v)
    iota16 = lax.broadcasted_iota(jnp.int32, (16,), 0)

    def issue_grp(t, b):
        off = (base + t) * _NG
        for h in (0, 16):
            ch = cidx_v[t, pl.ds(h, 16)]
            ggidx_v[b, pl.ds(h, 16)] = (ch >> 3) + off
        pltpu.async_copy(pre_hbm.at[ggidx_v.at[b]], grp_v.at[b], csem.at[b])

    issue_grp(0, 0)
    issue_grp(1, 1)

    def token_body(t, carry):
        b = lax.rem(t, 2)
        pltpu.make_async_copy(pre_hbm.at[pl.ds(0, _K)], grp_v.at[b],
                              csem.at[b]).wait()

        eyes = [eye_v[s, pl.ds(0, 16)] for s in range(8)]
        for jc in range(_K):
            vch = cidx_v[t, pl.ds((jc // 16) * 16, 16)]
            cidv = vch[jnp.full((16,), jc % 16, jnp.int32)]
            gi_v[jc] = cidv * 16 + iota16
            slot = cidv & 7
            kv = jnp.zeros((16,), jnp.float32)
            for s in range(8):
                kv = kv + eyes[s][slot] * grp_v[b, jc, pl.ds(s * 16, 16)]
            cand_v[jc] = kv
        pltpu.sync_copy(cand_v, cand_hbm.at[base + t])
        pltpu.sync_copy(gi_v, gi_hbm.at[base + t])

        @pl.when(t + 2 < _TPW)
        def _():
            issue_grp(t + 2, b)
        return carry

    lax.fori_loop(0, _TPW, token_body, 0)


@functools.cache
def _make_gather():
    return pl.kernel(
        _gather_body,
        mesh=plsc.VectorSubcoreMesh(core_axis_name="c", subcore_axis_name="s"),
        out_type=[
            jax.ShapeDtypeStruct((_S, _K, 16), jnp.float32),
            jax.ShapeDtypeStruct((_S, _K, 16), jnp.int32),
        ],
        scratch_types=[
            pltpu.VMEM((_TPW, _K), jnp.int32),
            pltpu.VMEM((2, _K), jnp.int32),
            pltpu.VMEM((2, _K, 128), jnp.float32),
            pltpu.VMEM((_K, 16), jnp.float32),
            pltpu.VMEM((_K, 16), jnp.int32),
            pltpu.VMEM((8, 16), jnp.float32),
            pltpu.SemaphoreType.DMA((2,)),
        ],
    )


def _select_body(cand_ref, gi_ref, vals_ref, gidx_ref):
    big = jnp.int32(_L)
    neg = jnp.float32(-jnp.inf)
    gi = gi_ref[...]
    for k in range(_K):
        buf = cand_ref[...]
        m = jnp.max(buf, axis=1)
        eq = buf == m[:, None]
        gmin = jnp.min(jnp.where(eq, gi, big), axis=1)
        vals_ref[:, k:k + 1] = m[:, None]
        gidx_ref[:, k:k + 1] = gmin[:, None]
        cand_ref[...] = jnp.where(gi == gmin[:, None], neg, buf)


_select = pl.pallas_call(
    _select_body,
    grid=(_S // _BS,),
    in_specs=[
        pl.BlockSpec((_BS, _NCAND), lambda i: (i, 0)),
        pl.BlockSpec((_BS, _NCAND), lambda i: (i, 0)),
    ],
    out_specs=[
        pl.BlockSpec((_BS, _K), lambda i: (i, 0)),
        pl.BlockSpec((_BS, _K), lambda i: (i, 0)),
    ],
    out_shape=[
        jax.ShapeDtypeStruct((_S, _K), jnp.float32),
        jax.ShapeDtypeStruct((_S, _K), jnp.int32),
    ],
)


def _decode_body(idx_hbm, vals_hbm, wdec_hbm, bdec_hbm, out_hbm,
                 idx_v, vals_v, rows_v, bdec_v, acc_v, sem):
    wid = lax.axis_index("s") * _NC + lax.axis_index("c")
    base = wid * _TPW
    pltpu.sync_copy(idx_hbm.at[pl.ds(base, _TPW)], idx_v)
    pltpu.sync_copy(vals_hbm.at[pl.ds(base, _TPW)], vals_v)
    pltpu.sync_copy(bdec_hbm, bdec_v)

    pltpu.async_copy(wdec_hbm.at[idx_v.at[0]], rows_v.at[0], sem.at[0])

    def token_body(t, carry):
        b = lax.rem(t, 2)

        @pl.when(t + 1 < _TPW)
        def _():
            pltpu.async_copy(wdec_hbm.at[idx_v.at[t + 1]], rows_v.at[1 - b],
                             sem.at[1 - b])

        pltpu.make_async_copy(wdec_hbm.at[pl.ds(0, _K)], rows_v.at[b],
                              sem.at[b]).wait()

        vch0 = vals_v[t, pl.ds(0, 16)]
        vch1 = vals_v[t, pl.ds(16, 16)]
        vbs = ([vch0[jnp.full((16,), i, jnp.int32)] for i in range(16)] +
               [vch1[jnp.full((16,), i, jnp.int32)] for i in range(16)])

        def c_body(c, carry2):
            off = c * 32
            a0 = bdec_v[pl.ds(off, 16)]
            b0 = bdec_v[pl.ds(off + 16, 16)]
            a1 = jnp.zeros((16,), jnp.float32)
            b1 = jnp.zeros((16,), jnp.float32)
            for k in range(_K):
                wa = rows_v[b, k, pl.ds(off, 16)]
                wb = rows_v[b, k, pl.ds(off + 16, 16)]
                if k % 2 == 0:
                    a0 = a0 + vbs[k] * wa
                    b0 = b0 + vbs[k] * wb
                else:
                    a1 = a1 + vbs[k] * wa
                    b1 = b1 + vbs[k] * wb
            acc_v[pl.ds(off, 16)] = a0 + a1
            acc_v[pl.ds(off + 16, 16)] = b0 + b1
            return carry2

        lax.fori_loop(0, _D // 32, c_body, 0)
        pltpu.sync_copy(acc_v, out_hbm.at[base + t])
        return carry

    lax.fori_loop(0, _TPW, token_body, 0)


@functools.cache
def _make_decode():
    return pl.kernel(
        _decode_body,
        mesh=plsc.VectorSubcoreMesh(core_axis_name="c", subcore_axis_name="s"),
        out_type=jax.ShapeDtypeStruct((_S, _D), jnp.float32),
        scratch_types=[
            pltpu.VMEM((_TPW, _K), jnp.int32),
            pltpu.VMEM((_TPW, _K), jnp.float32),
            pltpu.VMEM((2, _K, _D), jnp.float32),
            pltpu.VMEM((_D,), jnp.float32),
            pltpu.VMEM((_D,), jnp.float32),
            pltpu.SemaphoreType.DMA((2,)),
        ],
    )


def kernel(x, enc_W, enc_b, W_dec, b_dec):
    xs = x.reshape(_S, _D)
    pre3, cidx = _encode(xs, b_dec.reshape(1, _D), enc_W, enc_b.reshape(1, _L))
    pre_flat = pre3.reshape(_S * _NG, 128)
    eye = jnp.eye(8, 16, dtype=jnp.float32)
    cand, gi = _make_gather()(pre_flat, cidx, eye)
    vals, gidx = _select(cand.reshape(_S, _NCAND), gi.reshape(_S, _NCAND))
    out = _make_decode()(gidx, vals, W_dec, b_dec)
    return out.reshape(x.shape)
